# Initial kernel scaffold; baseline (speedup 1.0000x reference)
import jax
import jax.numpy as jnp
from jax.experimental import pallas as pl


def kernel(des, tweet, num_prop, cat_prop, edge_index, edge_type, Wd, bd, Wt, bt, Wn, bn, Wc, bc, Win, b_in, Wrel, Wroot, b_rgcn, Wo1, bo1, Wo2, bo2):
    raise NotImplementedError("write your pallas kernel here")


# speedup vs baseline: 5.9993x; 5.9993x over previous
import jax
import jax.numpy as jnp
from jax import lax
from jax.experimental import pallas as pl
from jax.experimental.pallas import tpu as pltpu
from jax.experimental.pallas import tpu_sc as plsc

LN = 128
RG = 8
NSUB = 16
NCORE = 2
B_RANGE = 22528
PAD_ROWS = 128
RSZ = B_RANGE + PAD_ROWS


def _leaky(x):
    return jnp.where(x >= 0, x, 0.01 * x)


def _enc_body(des_ref, tweet_ref, num_ref, cat_ref, Wd_ref, bd_ref, Wt_ref,
              bt_ref, Wn_ref, bn_ref, Wc_ref, bc_ref, Win_ref, bin_ref,
              x_ref):
    d = _leaky(des_ref[...] @ Wd_ref[...] + bd_ref[...])
    t = _leaky(tweet_ref[...] @ Wt_ref[...] + bt_ref[...])
    n = _leaky(num_ref[...] @ Wn_ref[...] + bn_ref[...])
    c = _leaky(cat_ref[...] @ Wc_ref[...] + bc_ref[...])
    x = jnp.concatenate((d, t, n, c), axis=1)
    x_ref[...] = _leaky(x @ Win_ref[...] + bin_ref[...])


def _encoder(des, tweet, num_prop, cat_prop, Wd, bd, Wt, bt, Wn, bn, Wc, bc,
             Win, b_in):
    N = des.shape[0]
    B = 2000
    grid = (N // B,)
    row = lambda i: (i, 0)
    rep = lambda i: (0, 0)
    spec = pl.BlockSpec
    emb = Win.shape[1]
    h = Wd.shape[1]
    return pl.pallas_call(
        _enc_body,
        grid=grid,
        in_specs=[
            spec((B, des.shape[1]), row),
            spec((B, tweet.shape[1]), row),
            spec((B, num_prop.shape[1]), row),
            spec((B, cat_prop.shape[1]), row),
            spec(Wd.shape, rep), spec((1, h), rep),
            spec(Wt.shape, rep), spec((1, h), rep),
            spec(Wn.shape, rep), spec((1, h), rep),
            spec(Wc.shape, rep), spec((1, h), rep),
            spec(Win.shape, rep), spec((1, emb), rep),
        ],
        out_specs=spec((B, emb), row),
        out_shape=jax.ShapeDtypeStruct((N, emb), jnp.float32),
    )(des, tweet, num_prop, cat_prop, Wd, bd.reshape(1, -1), Wt,
      bt.reshape(1, -1), Wn, bn.reshape(1, -1), Wc, bc.reshape(1, -1), Win,
      b_in.reshape(1, -1))


def _combine_body(x_ref, a0_ref, a1_ref, c0_ref, c1_ref, Wroot_ref, b_ref,
                  Wr0_ref, Wr1_ref, o_ref):
    x = x_ref[...]
    a0 = a0_ref[...] / jnp.maximum(c0_ref[...], 1.0)
    a1 = a1_ref[...] / jnp.maximum(c1_ref[...], 1.0)
    o_ref[...] = (x @ Wroot_ref[...] + b_ref[...] + a0 @ Wr0_ref[...]
                  + a1 @ Wr1_ref[...])


def _combine_mlp_body(x_ref, a0_ref, a1_ref, c0_ref, c1_ref, Wroot_ref, b_ref,
                      Wr0_ref, Wr1_ref, Wo1_ref, bo1_ref, Wo2_ref, bo2_ref,
                      o_ref):
    x = x_ref[...]
    a0 = a0_ref[...] / jnp.maximum(c0_ref[...], 1.0)
    a1 = a1_ref[...] / jnp.maximum(c1_ref[...], 1.0)
    h = (x @ Wroot_ref[...] + b_ref[...] + a0 @ Wr0_ref[...]
         + a1 @ Wr1_ref[...])
    h = _leaky(h @ Wo1_ref[...] + bo1_ref[...])
    o_ref[...] = h @ Wo2_ref[...] + bo2_ref[...]


def _combine(x, a0, a1, c0, c1, Wroot, b, Wr0, Wr1, mlp=None):
    N, emb = x.shape
    B = 5000
    grid = (N // B,)
    row = lambda i: (i, 0)
    rep = lambda i: (0, 0)
    spec = pl.BlockSpec
    in_specs = [
        spec((B, emb), row), spec((B, emb), row), spec((B, emb), row),
        spec((B, 1), row), spec((B, 1), row),
        spec(Wroot.shape, rep), spec((1, emb), rep),
        spec(Wr0.shape, rep), spec(Wr1.shape, rep),
    ]
    args = [x, a0, a1, c0, c1, Wroot, b.reshape(1, -1), Wr0, Wr1]
    if mlp is None:
        body = _combine_body
        out_dim = emb
    else:
        Wo1, bo1, Wo2, bo2 = mlp
        in_specs += [spec(Wo1.shape, rep), spec((1, emb), rep),
                     spec(Wo2.shape, rep), spec((1, Wo2.shape[1]), rep)]
        args += [Wo1, bo1.reshape(1, -1), Wo2, bo2.reshape(1, -1)]
        body = _combine_mlp_body
        out_dim = Wo2.shape[1]
    return pl.pallas_call(
        body,
        grid=grid,
        in_specs=in_specs,
        out_specs=spec((B, out_dim), row),
        out_shape=jax.ShapeDtypeStruct((N, out_dim), jnp.float32),
    )(*args)


def _zero_fill(buf, nrows, width):
    z = jnp.zeros((16,), jnp.float32)

    def body(i, _):
        for k in range(width // 16):
            buf[i, pl.ds(k * 16, 16)] = z
        return 0

    lax.fori_loop(0, nrows, body, 0)


def _sc_params():
    return pltpu.CompilerParams(use_tc_tiling_on_sc=False)


def _prep_kernel(RP, NRANGE):
    mesh = plsc.VectorSubcoreMesh(core_axis_name="c", subcore_axis_name="s")
    tr = RP // NSUB
    ng = tr // RG
    zrows = RSZ // NSUB
    drows = B_RANGE // NSUB

    def body(dst_hbm, typ_hbm, dstm_hbm, cnt_hbm, dstb, typb, lmb, onesb,
             zb, cnt_sp, sem):
        c = lax.axis_index("c")
        s = lax.axis_index("s")

        _zero_fill(zb, zrows, 16)

        def ones_body(i, _):
            onesb[i, pl.ds(0, 16)] = jnp.ones((16,), jnp.float32)
            return 0
        lax.fori_loop(0, LN, ones_body, 0)

        for m in range(NRANGE):
            base_node = m * B_RANGE
            pltpu.sync_copy(zb, cnt_sp.at[pl.ds(s * zrows, zrows)])
            plsc.subcore_barrier()

            def group(g, _):
                rbase = s * tr + g * RG
                pltpu.sync_copy(dst_hbm.at[pl.ds(rbase, RG)], dstb)
                pltpu.sync_copy(typ_hbm.at[pl.ds(rbase, RG)], typb)
                for j in range(RG):
                    for k in range(LN // 16):
                        t = typb[j, pl.ds(k * 16, 16)]
                        d = dstb[j, pl.ds(k * 16, 16)]
                        loc = d - base_node
                        ok = ((t == c) & (loc >= 0) & (loc < B_RANGE))
                        dummy = B_RANGE + (d & (PAD_ROWS - 1))
                        lmb[j, pl.ds(k * 16, 16)] = jnp.where(ok, loc, dummy)
                pltpu.sync_copy(lmb, dstm_hbm.at[c, m, pl.ds(rbase, RG)])
                descs = [
                    pltpu.async_copy(onesb, cnt_sp.at[lmb.at[j]], sem,
                                     add=True)
                    for j in range(RG)
                ]
                for dsc in descs:
                    dsc.wait()
                return 0

            lax.fori_loop(0, ng, group, 0)
            plsc.subcore_barrier()
            pltpu.sync_copy(
                cnt_sp.at[pl.ds(s * drows, drows)],
                cnt_hbm.at[c, pl.ds(base_node + s * drows, drows)])
            plsc.subcore_barrier()

    return pl.kernel(
        body,
        out_type=(
            jax.ShapeDtypeStruct((NCORE, NRANGE, RP, LN), jnp.int32),
            jax.ShapeDtypeStruct((NCORE, NRANGE * B_RANGE, 16), jnp.float32),
        ),
        mesh=mesh,
        compiler_params=_sc_params(),
        scratch_types=[
            pltpu.VMEM((RG, LN), jnp.int32),
            pltpu.VMEM((RG, LN), jnp.int32),
            pltpu.VMEM((RG, LN), jnp.int32),
            pltpu.VMEM((LN, 16), jnp.float32),
            pltpu.VMEM((RSZ // NSUB, 16), jnp.float32),
            pltpu.VMEM_SHARED((RSZ, 16), jnp.float32),
            pltpu.SemaphoreType.DMA,
        ],
    )


def _agg_kernel(N, RP, NRANGE, EMB):
    mesh = plsc.VectorSubcoreMesh(core_axis_name="c", subcore_axis_name="s")
    tr = RP // NSUB
    ng = tr // RG
    zrows = RSZ // NSUB
    drows = B_RANGE // NSUB

    def body(x_hbm, src_hbm, dstm_hbm, agg_hbm, idxs, idxl, rows, zb, acc,
             gsem, ssem):
        c = lax.axis_index("c")
        s = lax.axis_index("s")

        _zero_fill(zb, zrows, EMB)

        for m in range(NRANGE):
            pltpu.sync_copy(zb, acc.at[pl.ds(s * zrows, zrows)])
            plsc.subcore_barrier()

            def group(g, _):
                rbase = s * tr + g * RG
                pltpu.sync_copy(src_hbm.at[pl.ds(rbase, RG)], idxs)
                pltpu.sync_copy(dstm_hbm.at[c, m, pl.ds(rbase, RG)], idxl)
                gds = [
                    pltpu.async_copy(x_hbm.at[idxs.at[j]], rows.at[j], gsem)
                    for j in range(RG)
                ]
                for dsc in gds:
                    dsc.wait()
                sds = [
                    pltpu.async_copy(rows.at[j], acc.at[idxl.at[j]], ssem,
                                     add=True)
                    for j in range(RG)
                ]
                for dsc in sds:
                    dsc.wait()
                return 0

            lax.fori_loop(0, ng, group, 0)
            plsc.subcore_barrier()
            pltpu.sync_copy(
                acc.at[pl.ds(s * drows, drows)],
                agg_hbm.at[c, pl.ds(m * B_RANGE + s * drows, drows)])
            plsc.subcore_barrier()

    return pl.kernel(
        body,
        out_type=jax.ShapeDtypeStruct((NCORE, NRANGE * B_RANGE, EMB),
                                      jnp.float32),
        mesh=mesh,
        compiler_params=_sc_params(),
        scratch_types=[
            pltpu.VMEM((RG, LN), jnp.int32),
            pltpu.VMEM((RG, LN), jnp.int32),
            pltpu.VMEM((RG, LN, EMB), jnp.float32),
            pltpu.VMEM((RSZ // NSUB, EMB), jnp.float32),
            pltpu.VMEM_SHARED((RSZ, EMB), jnp.float32),
            pltpu.SemaphoreType.DMA,
            pltpu.SemaphoreType.DMA,
        ],
    )


def kernel(des, tweet, num_prop, cat_prop, edge_index, edge_type, Wd, bd, Wt,
           bt, Wn, bn, Wc, bc, Win, b_in, Wrel, Wroot, b_rgcn, Wo1, bo1, Wo2,
           bo2):
    N = des.shape[0]
    E = edge_index.shape[1]
    EMB = Win.shape[1]
    NRANGE = -(-N // B_RANGE)
    rows0 = -(-E // LN)
    RP = -(-rows0 // (NSUB * RG)) * (NSUB * RG)
    EP = RP * LN

    src2d = jnp.pad(edge_index[0], (0, EP - E)).reshape(RP, LN)
    dst2d = jnp.pad(edge_index[1], (0, EP - E)).reshape(RP, LN)
    typ2d = jnp.pad(edge_type, (0, EP - E), constant_values=2).reshape(RP, LN)

    x = _encoder(des, tweet, num_prop, cat_prop, Wd, bd, Wt, bt, Wn, bn, Wc,
                 bc, Win, b_in)

    dstm, cnt = _prep_kernel(RP, NRANGE)(dst2d, typ2d)
    c0 = cnt[0, :N, 0:1]
    c1 = cnt[1, :N, 0:1]

    agg_fn = _agg_kernel(N, RP, NRANGE, EMB)

    agg1 = agg_fn(x, src2d, dstm)
    x = _combine(x, agg1[0, :N], agg1[1, :N], c0, c1, Wroot, b_rgcn,
                 Wrel[0], Wrel[1])
    agg2 = agg_fn(x, src2d, dstm)
    out = _combine(x, agg2[0, :N], agg2[1, :N], c0, c1, Wroot, b_rgcn,
                   Wrel[0], Wrel[1], mlp=(Wo1, bo1, Wo2, bo2))
    return out


# speedup vs baseline: 9.3439x; 1.5575x over previous
import jax
import jax.numpy as jnp
from jax import lax
from jax.experimental import pallas as pl
from jax.experimental.pallas import tpu as pltpu
from jax.experimental.pallas import tpu_sc as plsc

LN = 128
RG = 8
NSUB = 16
NCORE = 2
B_RANGE = 22528
PAD_ROWS = 128
RSZ = B_RANGE + PAD_ROWS
SRC_BITS = 17


def _leaky(x):
    return jnp.where(x >= 0, x, 0.01 * x)


def _enc_body(des_ref, tweet_ref, num_ref, cat_ref, Wd_ref, bd_ref, Wt_ref,
              bt_ref, Wn_ref, bn_ref, Wc_ref, bc_ref, Win_ref, bin_ref,
              x_ref):
    d = _leaky(des_ref[...] @ Wd_ref[...] + bd_ref[...])
    t = _leaky(tweet_ref[...] @ Wt_ref[...] + bt_ref[...])
    n = _leaky(num_ref[...] @ Wn_ref[...] + bn_ref[...])
    c = _leaky(cat_ref[...] @ Wc_ref[...] + bc_ref[...])
    x = jnp.concatenate((d, t, n, c), axis=1)
    x_ref[...] = _leaky(x @ Win_ref[...] + bin_ref[...])


def _encoder(des, tweet, num_prop, cat_prop, Wd, bd, Wt, bt, Wn, bn, Wc, bc,
             Win, b_in):
    N = des.shape[0]
    B = 2000
    grid = (N // B,)
    row = lambda i: (i, 0)
    rep = lambda i: (0, 0)
    spec = pl.BlockSpec
    emb = Win.shape[1]
    h = Wd.shape[1]
    return pl.pallas_call(
        _enc_body,
        grid=grid,
        in_specs=[
            spec((B, des.shape[1]), row),
            spec((B, tweet.shape[1]), row),
            spec((B, num_prop.shape[1]), row),
            spec((B, cat_prop.shape[1]), row),
            spec(Wd.shape, rep), spec((1, h), rep),
            spec(Wt.shape, rep), spec((1, h), rep),
            spec(Wn.shape, rep), spec((1, h), rep),
            spec(Wc.shape, rep), spec((1, h), rep),
            spec(Win.shape, rep), spec((1, emb), rep),
        ],
        out_specs=spec((B, emb), row),
        out_shape=jax.ShapeDtypeStruct((N, emb), jnp.float32),
    )(des, tweet, num_prop, cat_prop, Wd, bd.reshape(1, -1), Wt,
      bt.reshape(1, -1), Wn, bn.reshape(1, -1), Wc, bc.reshape(1, -1), Win,
      b_in.reshape(1, -1))


def _combine_body(x_ref, a0_ref, a1_ref, c0_ref, c1_ref, Wroot_ref, b_ref,
                  Wr0_ref, Wr1_ref, o_ref):
    x = x_ref[...]
    a0 = a0_ref[...] / jnp.maximum(c0_ref[...], 1.0)
    a1 = a1_ref[...] / jnp.maximum(c1_ref[...], 1.0)
    o_ref[...] = (x @ Wroot_ref[...] + b_ref[...] + a0 @ Wr0_ref[...]
                  + a1 @ Wr1_ref[...])


def _combine_mlp_body(x_ref, a0_ref, a1_ref, c0_ref, c1_ref, Wroot_ref, b_ref,
                      Wr0_ref, Wr1_ref, Wo1_ref, bo1_ref, Wo2_ref, bo2_ref,
                      o_ref):
    x = x_ref[...]
    a0 = a0_ref[...] / jnp.maximum(c0_ref[...], 1.0)
    a1 = a1_ref[...] / jnp.maximum(c1_ref[...], 1.0)
    h = (x @ Wroot_ref[...] + b_ref[...] + a0 @ Wr0_ref[...]
         + a1 @ Wr1_ref[...])
    h = _leaky(h @ Wo1_ref[...] + bo1_ref[...])
    o_ref[...] = h @ Wo2_ref[...] + bo2_ref[...]


def _combine(x, a0, a1, c0, c1, Wroot, b, Wr0, Wr1, mlp=None):
    N, emb = x.shape
    B = 5000
    grid = (N // B,)
    row = lambda i: (i, 0)
    rep = lambda i: (0, 0)
    spec = pl.BlockSpec
    in_specs = [
        spec((B, emb), row), spec((B, emb), row), spec((B, emb), row),
        spec((B, 1), row), spec((B, 1), row),
        spec(Wroot.shape, rep), spec((1, emb), rep),
        spec(Wr0.shape, rep), spec(Wr1.shape, rep),
    ]
    args = [x, a0, a1, c0, c1, Wroot, b.reshape(1, -1), Wr0, Wr1]
    if mlp is None:
        body = _combine_body
        out_dim = emb
    else:
        Wo1, bo1, Wo2, bo2 = mlp
        in_specs += [spec(Wo1.shape, rep), spec((1, emb), rep),
                     spec(Wo2.shape, rep), spec((1, Wo2.shape[1]), rep)]
        args += [Wo1, bo1.reshape(1, -1), Wo2, bo2.reshape(1, -1)]
        body = _combine_mlp_body
        out_dim = Wo2.shape[1]
    return pl.pallas_call(
        body,
        grid=grid,
        in_specs=in_specs,
        out_specs=spec((B, out_dim), row),
        out_shape=jax.ShapeDtypeStruct((N, out_dim), jnp.float32),
    )(*args)


def _zero_fill(buf, nrows, width):
    z = jnp.zeros((16,), jnp.float32)

    def body(i, _):
        for k in range(width // 16):
            buf[i, pl.ds(k * 16, 16)] = z
        return 0

    lax.fori_loop(0, nrows, body, 0)


def _sc_params():
    return pltpu.CompilerParams(use_tc_tiling_on_sc=False,
                                needs_layout_passes=False)


def _part_kernel(RP, NRANGE):
    mesh = plsc.VectorSubcoreMesh(core_axis_name="c", subcore_axis_name="s")
    tr = RP // NSUB
    ng = tr // RG
    zrows = RSZ // NSUB
    drows = B_RANGE // NSUB

    def body(src_hbm, dst_hbm, typ_hbm, pcodes, ngrp, cnt_hbm, srcb, dstb,
             typb, runb, ngb, cb, lb, onesb, zb, cnt_sp, sem):
        c = lax.axis_index("c")
        s = lax.axis_index("s")
        i32 = jnp.int32
        iota = lax.iota(i32, 16)
        dummy_code = jnp.left_shift(
            B_RANGE + ((iota * 8 + s * 2) & (PAD_ROWS - 1)), SRC_BITS)

        def grp1(g, carry):
            fills, rows = carry
            rbase = s * tr + g * RG
            pltpu.sync_copy(src_hbm.at[pl.ds(rbase, RG)], srcb)
            pltpu.sync_copy(dst_hbm.at[pl.ds(rbase, RG)], dstb)
            pltpu.sync_copy(typ_hbm.at[pl.ds(rbase, RG)], typb)
            for j in range(RG):
                for k in range(LN // 16):
                    sl = pl.ds(k * 16, 16)
                    sv = srcb[j, sl]
                    dv = dstb[j, sl]
                    tv = typb[j, sl]
                    mv = ((dv >= B_RANGE).astype(i32)
                          + (dv >= 2 * B_RANGE).astype(i32))
                    code = sv | jnp.left_shift(dv - mv * B_RANGE, SRC_BITS)
                    keep = tv == c
                    new_f = []
                    new_r = []
                    for m in range(NRANGE):
                        mk = keep & (mv == m)
                        fill = fills[m]
                        row = rows[m]
                        plsc.store_compressed(
                            runb.at[m, 0, pl.ds(fill, 16)], code, mask=mk)
                        fill = fill + jnp.sum(mk.astype(i32))
                        do = fill >= LN

                        @pl.when(do)
                        def _(m=m, row=row):
                            pltpu.sync_copy(
                                runb.at[m, :, pl.ds(0, LN)],
                                pcodes.at[c, m, pl.ds(s * tr + row, 1)])
                            tail = runb[m, 0, pl.ds(LN, 16)]
                            runb[m, 0, pl.ds(0, 16)] = tail

                        new_f.append(jnp.where(do, fill - LN, fill))
                        new_r.append(jnp.where(do, row + 1, row))
                    fills = tuple(new_f)
                    rows = tuple(new_r)
            return fills, rows

        zero = jnp.zeros((), i32)
        fills, rows = lax.fori_loop(
            0, ng, grp1, ((zero, zero, zero), (zero, zero, zero)))

        for m in range(NRANGE):
            fill = fills[m]
            row = rows[m]
            for w in range(LN // 16):
                runb[m, 0, pl.ds(fill + w * 16, 16)] = dummy_code

            @pl.when(fill > 0)
            def _(m=m, row=row):
                pltpu.sync_copy(runb.at[m, :, pl.ds(0, LN)],
                                pcodes.at[c, m, pl.ds(s * tr + row, 1)])

            row = row + (fill > 0).astype(i32)
            for w in range(LN // 16):
                runb[m, 0, pl.ds(w * 16, 16)] = dummy_code
            npad = (-row) % RG

            def padloop(i, row=row, m=m):
                pltpu.sync_copy(runb.at[m, :, pl.ds(0, LN)],
                                pcodes.at[c, m, pl.ds(s * tr + row + i, 1)])
                return 0

            lax.fori_loop(0, npad, lambda i, _, m=m, row=row: (
                pltpu.sync_copy(
                    runb.at[m, :, pl.ds(0, LN)],
                    pcodes.at[c, m, pl.ds(s * tr + row + i, 1)]), 0)[1], 0)
            ngroups = (row + npad) // RG
            ngb[0, pl.ds(0, 16)] = jnp.broadcast_to(ngroups, (16,)).astype(i32)
            pltpu.sync_copy(ngb.at[0], ngrp.at[c, m, s])
            fills = fills[:m] + (ngroups,) + fills[m + 1:]

        def ones_body(i, _):
            onesb[i, pl.ds(0, 16)] = jnp.ones((16,), jnp.float32)
            return 0
        lax.fori_loop(0, LN, ones_body, 0)
        _zero_fill(zb, zrows, 16)

        for m in range(NRANGE):
            my_groups = fills[m]
            pltpu.sync_copy(zb, cnt_sp.at[pl.ds(s * zrows, zrows)])
            plsc.subcore_barrier()

            def grp2(g, _, m=m):
                rbase = s * tr + g * RG
                pltpu.sync_copy(pcodes.at[c, m, pl.ds(rbase, RG)], cb)
                for j in range(RG):
                    for k in range(LN // 16):
                        sl = pl.ds(k * 16, 16)
                        lb[j, sl] = lax.shift_right_logical(cb[j, sl],
                                                            SRC_BITS)
                descs = [
                    pltpu.async_copy(onesb, cnt_sp.at[lb.at[j]], sem,
                                     add=True)
                    for j in range(RG)
                ]
                for dsc in descs:
                    dsc.wait()
                return 0

            lax.fori_loop(0, my_groups, grp2, 0)
            plsc.subcore_barrier()
            pltpu.sync_copy(
                cnt_sp.at[pl.ds(s * drows, drows)],
                cnt_hbm.at[c, pl.ds(m * B_RANGE + s * drows, drows)])
            plsc.subcore_barrier()

    return pl.kernel(
        body,
        out_type=(
            jax.ShapeDtypeStruct((NCORE, NRANGE, RP, LN), jnp.int32),
            jax.ShapeDtypeStruct((NCORE, NRANGE, NSUB, 16), jnp.int32),
            jax.ShapeDtypeStruct((NCORE, NRANGE * B_RANGE, 16), jnp.float32),
        ),
        mesh=mesh,
        compiler_params=_sc_params(),
        scratch_types=[
            pltpu.VMEM((RG, LN), jnp.int32),
            pltpu.VMEM((RG, LN), jnp.int32),
            pltpu.VMEM((RG, LN), jnp.int32),
            pltpu.VMEM((3, 1, 2 * LN + 16), jnp.int32),
            pltpu.VMEM((1, 16), jnp.int32),
            pltpu.VMEM((RG, LN), jnp.int32),
            pltpu.VMEM((RG, LN), jnp.int32),
            pltpu.VMEM((LN, 16), jnp.float32),
            pltpu.VMEM((RSZ // NSUB, 16), jnp.float32),
            pltpu.VMEM_SHARED((RSZ, 16), jnp.float32),
            pltpu.SemaphoreType.DMA,
        ],
    )


def _agg_kernel(N, RP, NRANGE, EMB):
    mesh = plsc.VectorSubcoreMesh(core_axis_name="c", subcore_axis_name="s")
    tr = RP // NSUB
    zrows = RSZ // NSUB
    drows = B_RANGE // NSUB

    def body(x_hbm, pcodes, ngrp, agg_hbm, cb, sb, lb, rows, zb, nb, acc,
             gsem, ssem):
        c = lax.axis_index("c")
        s = lax.axis_index("s")

        _zero_fill(zb, zrows, EMB)

        for m in range(NRANGE):
            pltpu.sync_copy(ngrp.at[c, m, s], nb)
            my_groups = jnp.max(nb[pl.ds(0, 16)])
            pltpu.sync_copy(zb, acc.at[pl.ds(s * zrows, zrows)])
            plsc.subcore_barrier()

            def group(g, _, m=m):
                rbase = s * tr + g * RG
                pltpu.sync_copy(pcodes.at[c, m, pl.ds(rbase, RG)], cb)
                for j in range(RG):
                    for k in range(LN // 16):
                        sl = pl.ds(k * 16, 16)
                        code = cb[j, sl]
                        sb[j, sl] = code & ((1 << SRC_BITS) - 1)
                        lb[j, sl] = lax.shift_right_logical(code, SRC_BITS)
                gds = [
                    pltpu.async_copy(x_hbm.at[sb.at[j]], rows.at[j], gsem)
                    for j in range(RG)
                ]
                for dsc in gds:
                    dsc.wait()
                sds = [
                    pltpu.async_copy(rows.at[j], acc.at[lb.at[j]], ssem,
                                     add=True)
                    for j in range(RG)
                ]
                for dsc in sds:
                    dsc.wait()
                return 0

            lax.fori_loop(0, my_groups, group, 0)
            plsc.subcore_barrier()
            pltpu.sync_copy(
                acc.at[pl.ds(s * drows, drows)],
                agg_hbm.at[c, pl.ds(m * B_RANGE + s * drows, drows)])
            plsc.subcore_barrier()

    return pl.kernel(
        body,
        out_type=jax.ShapeDtypeStruct((NCORE, NRANGE * B_RANGE, EMB),
                                      jnp.float32),
        mesh=mesh,
        compiler_params=_sc_params(),
        scratch_types=[
            pltpu.VMEM((RG, LN), jnp.int32),
            pltpu.VMEM((RG, LN), jnp.int32),
            pltpu.VMEM((RG, LN), jnp.int32),
            pltpu.VMEM((RG, LN, EMB), jnp.float32),
            pltpu.VMEM((RSZ // NSUB, EMB), jnp.float32),
            pltpu.VMEM((16,), jnp.int32),
            pltpu.VMEM_SHARED((RSZ, EMB), jnp.float32),
            pltpu.SemaphoreType.DMA,
            pltpu.SemaphoreType.DMA,
        ],
    )


def kernel(des, tweet, num_prop, cat_prop, edge_index, edge_type, Wd, bd, Wt,
           bt, Wn, bn, Wc, bc, Win, b_in, Wrel, Wroot, b_rgcn, Wo1, bo1, Wo2,
           bo2):
    N = des.shape[0]
    E = edge_index.shape[1]
    EMB = Win.shape[1]
    NRANGE = -(-N // B_RANGE)
    rows0 = -(-E // LN)
    RP = -(-rows0 // (NSUB * RG)) * (NSUB * RG)
    EP = RP * LN

    src2d = jnp.pad(edge_index[0], (0, EP - E)).reshape(RP, LN)
    dst2d = jnp.pad(edge_index[1], (0, EP - E)).reshape(RP, LN)
    typ2d = jnp.pad(edge_type, (0, EP - E), constant_values=2).reshape(RP, LN)

    x = _encoder(des, tweet, num_prop, cat_prop, Wd, bd, Wt, bt, Wn, bn, Wc,
                 bc, Win, b_in)

    pcodes, ngrp, cnt = _part_kernel(RP, NRANGE)(src2d, dst2d, typ2d)
    c0 = cnt[0, :N, 0:1]
    c1 = cnt[1, :N, 0:1]

    agg_fn = _agg_kernel(N, RP, NRANGE, EMB)

    agg1 = agg_fn(x, pcodes, ngrp)
    x = _combine(x, agg1[0, :N], agg1[1, :N], c0, c1, Wroot, b_rgcn,
                 Wrel[0], Wrel[1])
    agg2 = agg_fn(x, pcodes, ngrp)
    out = _combine(x, agg2[0, :N], agg2[1, :N], c0, c1, Wroot, b_rgcn,
                   Wrel[0], Wrel[1], mlp=(Wo1, bo1, Wo2, bo2))
    return out
